# Initial kernel scaffold; baseline (speedup 1.0000x reference)
import jax
import jax.numpy as jnp
from jax.experimental import pallas as pl


def kernel(x1, x2, edge_index1, edge_index2, W, b, w1, b1, w2, b2):
    raise NotImplementedError("write your pallas kernel here")


# speedup vs baseline: 5.1374x; 5.1374x over previous
import functools

import jax
import jax.numpy as jnp
from jax import lax
from jax.experimental import pallas as pl
from jax.experimental.pallas import tpu as pltpu
from jax.experimental.pallas import tpu_sc as plsc

_N = 10000
_D = 128
_E = 320000
_K = 15
_H = 16

_RPAD = 10240
_CHUNK = 128
_NCH = 157
_EPG = 16 * _NCH * _CHUNK
_ROWS_PER_TILE = _RPAD // 16

_LIN_BLK = 800


def _linear_body(x_ref, w_ref, b_ref, o_ref):
    o_ref[...] = jnp.dot(x_ref[...], w_ref[...],
                         preferred_element_type=jnp.float32) + b_ref[...]


_linear = pl.pallas_call(
    _linear_body,
    grid=(2 * _N // _LIN_BLK,),
    in_specs=[
        pl.BlockSpec((_LIN_BLK, _D), lambda i: (i, 0)),
        pl.BlockSpec((_D, _D), lambda i: (0, 0)),
        pl.BlockSpec((1, _D), lambda i: (0, 0)),
    ],
    out_specs=pl.BlockSpec((_LIN_BLK, _D), lambda i: (i, 0)),
    out_shape=jax.ShapeDtypeStruct((2 * _N, _D), jnp.float32),
)


_HALF = _RPAD // 2
_GARB = 1024
_ZROWS = _HALF // 16


def _segsum_body(h_hbm, src_hbm, dst_hbm, out_hbm,
                 src_t, dst_t, buf_a, buf_b, idx_r, agg_sh, sem_a, sem_b):
    c = lax.axis_index("c")
    s = lax.axis_index("s")
    lo = c * _HALF

    zero16 = jnp.zeros((16,), jnp.float32)

    def _start(j, buf, sem):
        pltpu.async_copy(h_hbm.at[src_t.at[j]], buf, sem)

    def _wait(buf, sem):
        pltpu.make_async_copy(h_hbm.at[src_t.at[0]], buf, sem).wait()

    def _redirect(j):
        for v in range(8):
            dv = dst_t[j, pl.ds(v * 16, 16)]
            local = dv - lo
            ok = (local >= 0) & (local < _HALF)
            garb = _HALF + ((j * _CHUNK + v * 16 + lax.iota(jnp.int32, 16))
                            & (_GARB - 1))
            idx_r[pl.ds(v * 16, 16)] = jnp.where(ok, local, garb)

    def _scat(j, buf):
        _redirect(j)
        pltpu.sync_copy(buf, agg_sh.at[idx_r], add=True)

    for g in (0, 1):
        pltpu.sync_copy(src_hbm.at[g, s], src_t)
        pltpu.sync_copy(dst_hbm.at[g, s], dst_t)

        def _zb(t, carry):
            buf_a[t // (_D // 16), pl.ds((t % (_D // 16)) * 16, 16)] = zero16
            return carry

        lax.fori_loop(0, _CHUNK * (_D // 16), _zb, 0)

        zbase = s * _ZROWS
        for off in range(0, _ZROWS, _CHUNK):
            sz = min(_CHUNK, _ZROWS - off)
            pltpu.sync_copy(buf_a.at[pl.ds(0, sz)],
                            agg_sh.at[pl.ds(zbase + off, sz)])
        plsc.subcore_barrier()

        _start(0, buf_a, sem_a)

        def _body(m, carry):
            _wait(buf_a, sem_a)
            _start(2 * m + 1, buf_b, sem_b)
            _scat(2 * m, buf_a)
            _wait(buf_b, sem_b)
            _start(2 * m + 2, buf_a, sem_a)
            _scat(2 * m + 1, buf_b)
            return carry

        lax.fori_loop(0, (_NCH - 1) // 2, _body, 0)
        _wait(buf_a, sem_a)
        _scat(_NCH - 1, buf_a)

        plsc.subcore_barrier()
        pltpu.sync_copy(agg_sh.at[pl.ds(zbase, _ZROWS)],
                        out_hbm.at[g, pl.ds(lo + zbase, _ZROWS)])
        plsc.subcore_barrier()


_segsum = functools.partial(
    pl.kernel,
    out_type=jax.ShapeDtypeStruct((2, _RPAD, _D), jnp.float32),
    mesh=plsc.VectorSubcoreMesh(core_axis_name="c", subcore_axis_name="s"),
    scratch_types=[
        pltpu.VMEM((_NCH, _CHUNK), jnp.int32),
        pltpu.VMEM((_NCH, _CHUNK), jnp.int32),
        pltpu.VMEM((_CHUNK, _D), jnp.float32),
        pltpu.VMEM((_CHUNK, _D), jnp.float32),
        pltpu.VMEM((_CHUNK,), jnp.int32),
        pltpu.VMEM_SHARED((_HALF + _GARB, _D), jnp.float32),
        pltpu.SemaphoreType.DMA,
        pltpu.SemaphoreType.DMA,
    ],
)(_segsum_body)


def _final_body(agg_ref, w1_ref, b1_ref, w2_ref, b2_ref, o_ref, sim_ref):
    i = pl.program_id(0)
    a = agg_ref[...]
    h1 = jnp.maximum(a[0], 0.0)
    h2 = jnp.maximum(a[1], 0.0)
    num = jnp.sum(h1 * h2, axis=1, keepdims=True)
    n1 = jnp.sqrt(jnp.sum(h1 * h1, axis=1, keepdims=True))
    n2 = jnp.sqrt(jnp.sum(h2 * h2, axis=1, keepdims=True))
    sim = num / (jnp.maximum(n1, 1e-8) * jnp.maximum(n2, 1e-8))
    grow = i * _CHUNK + lax.broadcasted_iota(jnp.int32, (_CHUNK, 1), 0)
    sim = jnp.where(grow < _N, sim, jnp.float32(jnp.inf))
    sim_ref[pl.ds(i * _CHUNK, _CHUNK), :] = sim

    @pl.when(i == _RPAD // _CHUNK - 1)
    def _tail():
        scur = sim_ref[...]
        ridx = lax.broadcasted_iota(jnp.int32, (_RPAD, 1), 0)
        ms = []
        for _ in range(_K):
            m = jnp.min(scur)
            p = jnp.min(jnp.where(scur == m, ridx, jnp.int32(2**30)))
            scur = jnp.where(ridx == p, jnp.float32(jnp.inf), scur)
            ms.append(m)
        acc = b2_ref[0, 0]
        for j in range(_H):
            t = b1_ref[0, j]
            for k in range(_K):
                t = t + ms[k] * w1_ref[k, j]
            acc = acc + jnp.maximum(t, 0.0) * w2_ref[j, 0]
        o_ref[...] = jnp.full((1, 1), acc, jnp.float32)


_final = pl.pallas_call(
    _final_body,
    grid=(_RPAD // _CHUNK,),
    in_specs=[
        pl.BlockSpec((2, _CHUNK, _D), lambda i: (0, i, 0)),
        pl.BlockSpec(memory_space=pltpu.SMEM),
        pl.BlockSpec(memory_space=pltpu.SMEM),
        pl.BlockSpec(memory_space=pltpu.SMEM),
        pl.BlockSpec(memory_space=pltpu.SMEM),
    ],
    out_specs=pl.BlockSpec((1, 1), lambda i: (0, 0)),
    out_shape=jax.ShapeDtypeStruct((1, 1), jnp.float32),
    scratch_shapes=[pltpu.VMEM((_RPAD, 1), jnp.float32)],
)


def kernel(x1, x2, edge_index1, edge_index2, W, b, w1, b1, w2, b2):
    x_all = jnp.concatenate([x1, x2], axis=0)
    h = _linear(x_all, W, b.reshape(1, _D))

    npad = _EPG - _E
    pad_i = jnp.arange(npad, dtype=jnp.int32)
    srcs, dsts = [], []
    for g, ei in ((0, edge_index1), (1, edge_index2)):
        src = jnp.concatenate([ei[0], pad_i % _N]) + g * _N
        dst = jnp.concatenate([ei[1], _N + pad_i % (_RPAD - _N)])
        srcs.append(src.reshape(16, _NCH, _CHUNK))
        dsts.append(dst.reshape(16, _NCH, _CHUNK))

    agg = _segsum(h, jnp.stack(srcs), jnp.stack(dsts))
    return _final(agg, w1, b1.reshape(1, _H), w2, b2.reshape(1, 1))
